# baseline (speedup 1.0000x reference)
import functools

import jax
import jax.numpy as jnp
from jax import lax
from jax.experimental import pallas as pl
from jax.experimental.pallas import tpu as pltpu
from jax.experimental.pallas import tpu_sc as plsc

_N = 10000
_E = 320000
_D = 64
_NC = 2
_NS = 16
_ROW = 125
_GROUP = 8
_GR = _GROUP * _ROW
_EB = 2560

_mesh = lambda: plsc.VectorSubcoreMesh(core_axis_name="c", subcore_axis_name="s")
_SC_PARAMS = pltpu.CompilerParams(use_tc_tiling_on_sc=False)


def _silu(x):
    return x * jax.nn.sigmoid(x)


def _k0_gather(x2, dst2, src2):
    RPT = _E // _NS
    GP = 4
    GRH = GP * _ROW
    NG = RPT // GRH

    @functools.partial(
        pl.kernel,
        mesh=_mesh(),
        compiler_params=_SC_PARAMS,
        out_type=(
            jax.ShapeDtypeStruct((_E, _D), jnp.float32),
            jax.ShapeDtypeStruct((_E, _D), jnp.float32),
        ),
        scratch_types=[
            pltpu.VMEM((GP, _ROW), jnp.int32),
            pltpu.VMEM((GP, _ROW), jnp.int32),
            pltpu.VMEM((GRH, _D), jnp.float32),
            pltpu.VMEM((GRH, _D), jnp.float32),
            pltpu.SemaphoreType.DMA,
            pltpu.SemaphoreType.DMA,
        ],
    )
    def k(x_hbm, dstr_hbm, srcr_hbm, outd_hbm, outs_hbm, idx0_v, idx1_v,
          rows0_v, rows1_v, sem0, sem1):
        c = lax.axis_index("c")
        s = lax.axis_index("s")
        rbase = s * (RPT // _ROW)

        def run(idx_hbm, out_hbm):
            sets = ((idx0_v, rows0_v, sem0), (idx1_v, rows1_v, sem1))

            def fire(g, st):
                idx_v, rows_v, sem = st
                pltpu.sync_copy(idx_hbm.at[pl.ds(rbase + g * GP, GP)], idx_v)
                for j in range(GP):
                    pltpu.async_copy(x_hbm.at[idx_v.at[j]],
                                     rows_v.at[pl.ds(j * _ROW, _ROW)], sem)

            def drain_write(g, st):
                idx_v, rows_v, sem = st
                for j in range(GP):
                    pltpu.make_async_copy(
                        x_hbm.at[idx_v.at[j]],
                        rows_v.at[pl.ds(j * _ROW, _ROW)], sem).wait()
                pltpu.sync_copy(rows_v,
                                out_hbm.at[pl.ds(s * RPT + g * GRH, GRH)])

            fire(0, sets[0])

            def body(h, carry):
                g0 = 2 * h
                fire(g0 + 1, sets[1])
                drain_write(g0, sets[0])

                @pl.when(g0 + 2 < NG)
                def _():
                    fire(g0 + 2, sets[0])
                drain_write(g0 + 1, sets[1])
                return carry
            lax.fori_loop(0, NG // 2, body, 0)

        @pl.when(c == 0)
        def _():
            run(dstr_hbm, outd_hbm)

        @pl.when(c == 1)
        def _():
            run(srcr_hbm, outs_hbm)

    return k(x2, dst2, src2)


def _sc_scatter_add(dst2, vals, width):
    EPT = _E // (_NC * _NS)
    group = 8 if width <= 16 else 4
    gr = group * _ROW
    NG = EPT // gr
    NZR = _N // _NS
    chunks = []
    off = 0
    while off < NZR:
        sz = min(gr, NZR - off)
        chunks.append((off, sz))
        off += sz

    @functools.partial(
        pl.kernel,
        mesh=_mesh(),
        compiler_params=_SC_PARAMS,
        out_type=(
            jax.ShapeDtypeStruct((_N, width), jnp.float32),
            jax.ShapeDtypeStruct((_N, width), jnp.float32),
        ),
        scratch_types=[
            pltpu.VMEM_SHARED((_N, width), jnp.float32),
            pltpu.VMEM((group, _ROW), jnp.int32),
            pltpu.VMEM((gr, width), jnp.float32),
            pltpu.SemaphoreType.DMA,
        ],
    )
    def k(dst_hbm, vals_hbm, out0_hbm, out1_hbm, acc_sh, idx_v, rows_v, sem):
        c = lax.axis_index("c")
        s = lax.axis_index("s")

        def zero_row(i, carry):
            for kk in range(width // 16):
                rows_v[i, pl.ds(kk * 16, 16)] = jnp.zeros((16,), jnp.float32)
            return carry
        lax.fori_loop(0, min(gr, NZR), zero_row, 0)
        for off, sz in chunks:
            pltpu.sync_copy(rows_v.at[pl.ds(0, sz)],
                            acc_sh.at[pl.ds(s * NZR + off, sz)])
        plsc.subcore_barrier()

        wid = c * _NS + s
        rbase = wid * (EPT // _ROW)

        def body(g, carry):
            pltpu.sync_copy(dst_hbm.at[pl.ds(rbase + g * group, group)],
                            idx_v)
            pltpu.sync_copy(vals_hbm.at[pl.ds(wid * EPT + g * gr, gr)],
                            rows_v)
            descs = []
            for j in range(group):
                descs.append(pltpu.async_copy(
                    rows_v.at[pl.ds(j * _ROW, _ROW)],
                    acc_sh.at[idx_v.at[j]], sem, add=True))
            for d in descs:
                d.wait()
            return carry
        lax.fori_loop(0, NG, body, 0)
        plsc.subcore_barrier()

        for off, sz in chunks:
            pltpu.sync_copy(acc_sh.at[pl.ds(s * NZR + off, sz)],
                            rows_v.at[pl.ds(0, sz)])

            @pl.when(c == 0)
            def _(off=off, sz=sz):
                pltpu.sync_copy(rows_v.at[pl.ds(0, sz)],
                                out0_hbm.at[pl.ds(s * NZR + off, sz)])

            @pl.when(c == 1)
            def _(off=off, sz=sz):
                pltpu.sync_copy(rows_v.at[pl.ds(0, sz)],
                                out1_hbm.at[pl.ds(s * NZR + off, sz)])

    return k(dst2, vals)


def _k2b_gather2(s0, s1, dst2):
    EPT = _E // (_NC * _NS)
    NG = EPT // _GR

    @functools.partial(
        pl.kernel,
        mesh=_mesh(),
        compiler_params=_SC_PARAMS,
        out_type=jax.ShapeDtypeStruct((_E, 16), jnp.float32),
        scratch_types=[
            pltpu.VMEM((_GROUP, _ROW), jnp.int32),
            pltpu.VMEM((_GR, 16), jnp.float32),
            pltpu.VMEM((_GR, 16), jnp.float32),
            pltpu.SemaphoreType.DMA,
        ],
    )
    def k(s0_hbm, s1_hbm, dstr_hbm, o_hbm, idx_v, r0_v, r1_v, sem):
        c = lax.axis_index("c")
        s = lax.axis_index("s")
        wid = c * _NS + s
        rbase = wid * (EPT // _ROW)

        def body(g, carry):
            pltpu.sync_copy(dstr_hbm.at[pl.ds(rbase + g * _GROUP, _GROUP)],
                            idx_v)
            descs = []
            for j in range(_GROUP):
                descs.append(pltpu.async_copy(
                    s0_hbm.at[idx_v.at[j]],
                    r0_v.at[pl.ds(j * _ROW, _ROW)], sem))
                descs.append(pltpu.async_copy(
                    s1_hbm.at[idx_v.at[j]],
                    r1_v.at[pl.ds(j * _ROW, _ROW)], sem))
            for d in descs:
                d.wait()

            def addrow(i, carry2):
                for kk in range(4):
                    row = i * 4 + kk
                    r0_v[row, :] = r0_v[row, :] + r1_v[row, :]
                return carry2
            lax.fori_loop(0, _GR // 4, addrow, 0)
            pltpu.sync_copy(r0_v, o_hbm.at[pl.ds(wid * EPT + g * _GR, _GR)])
            return carry
        lax.fori_loop(0, NG, body, 0)

    return k(s0, s1, dst2)


def _dot(a, b):
    return jnp.dot(a, b, preferred_element_type=jnp.float32)


def _cat(*xs):
    return jnp.concatenate(xs, axis=1)


def _m0_in_paired(xdP, xsP):
    return _cat(xdP[:, 0:16], xsP[:, 0:16], xdP[:, 32:48], xsP[:, 32:48],
                xdP[:, 64:80], xsP[:, 64:80], xdP[:, 96:112], xsP[:, 96:112])


def _k1_body(edP, xdP, xsP, Wr1P, br1P, Wr2P, br2P, Wg01P, bg01P, Wm0aP,
             S2P, lng4P, lnb4P, adotP, SselP, ea_ref, rad_ref):
    h = _silu(_dot(edP[...], Wr1P[...]) + br1P[...])
    radP = _silu(_dot(h, Wr2P[...]) + br2P[...])
    rad_ref[...] = radP
    g01P = _dot(radP, Wg01P[...]) + bg01P[...]
    gate0P = _cat(g01P[:, 0:64], g01P[:, 96:160])
    tP = _m0_in_paired(xdP[...], xsP[...]) * gate0P
    a1P = _dot(tP, Wm0aP[...])
    statsP = _dot(_cat(a1P, a1P * a1P), S2P[...])
    muP = _cat(statsP[:, 0:64], statsP[:, 128:192])
    msqP = _cat(statsP[:, 64:128], statsP[:, 192:256])
    varP = msqP - muP * muP
    xnP = (a1P - muP) * lax.rsqrt(varP + 1e-5) * lng4P[...] + lnb4P[...]
    slP = 0.8 * xnP * jax.nn.sigmoid(xnP) + 0.2 * xnP
    eaP = jnp.exp(_dot(slP * adotP[...], SselP[...]))
    z12 = jnp.zeros((eaP.shape[0], 12), jnp.float32)
    ea_ref[...] = _cat(eaP[:, 0:4], z12, eaP[:, 4:8], z12)


def _k3_body(xdP, xsP, radP, eab, sdP, Wg01P, bg01P, Wm0vP, wrepP, W1cP,
             W2m0P, W2cP, WpcP, out_ref):
    xdP_, xsP_, sdP_, eab_ = xdP[...], xsP[...], sdP[...], eab[...]
    g01P = _dot(radP[...], Wg01P[...]) + bg01P[...]
    gate0P = _cat(g01P[:, 0:64], g01P[:, 96:160])
    tP = _m0_in_paired(xdP_, xsP_) * gate0P
    m0fP = _dot(tP, Wm0vP[...])
    eaP = _cat(eab_[:, 0:4], eab_[:, 16:20])
    w = eaP / (_cat(sdP_[:, 0:4], sdP_[:, 16:20]) + 1e-9)
    wtP = _dot(w, wrepP[...])
    m01P = _cat(m0fP[:, 32:64], m0fP[:, 128:160])
    gP = _cat(m0fP[:, 64:96], m0fP[:, 160:192])
    sgP = jax.nn.sigmoid(gP)
    g1e0 = g01P[:, 64:96]
    g1e1 = g01P[:, 160:192]
    rimP = _cat(xdP_[:, 16:32], xsP_[:, 16:32], xdP_[:, 48:64],
                xsP_[:, 48:64], xdP_[:, 80:96], xsP_[:, 80:96],
                xdP_[:, 112:128], xsP_[:, 112:128]) * _cat(
                    g1e0, g1e0, g1e1, g1e1)
    o1P = _dot(rimP, W1cP[...])
    m0in2P = _cat(gP[:, 0:32] * sgP[:, 0:32], m01P[:, 0:32] * sgP[:, 0:32],
                  gP[:, 32:64] * sgP[:, 32:64],
                  m01P[:, 32:64] * sgP[:, 32:64])
    m0o2P = _dot(m0in2P, W2m0P[...])
    r2i2P = _cat(o1P[:, 0:32] * sgP[:, 0:32], o1P[:, 32:64] * sgP[:, 0:32],
                 o1P[:, 64:96] * sgP[:, 32:64],
                 o1P[:, 96:128] * sgP[:, 32:64])
    ocP = _dot(r2i2P, W2cP[...])
    w0 = wtP[:, 0:64]
    w1 = wtP[:, 64:128]
    vcatP = _cat(m0o2P[:, 0:64] * w0, ocP[:, 0:64] * w0,
                 m0o2P[:, 64:128] * w0, ocP[:, 64:128] * w0,
                 m0o2P[:, 128:192] * w1, ocP[:, 128:192] * w1,
                 m0o2P[:, 192:256] * w1, ocP[:, 192:256] * w1)
    out_ref[...] = _dot(vcatP, WpcP[...])


def _k5_body(n0, n1, bias, out_ref):
    out_ref[...] = n0[...] + n1[...] + bias[...]


def _blk_spec(rows):
    return pl.BlockSpec((rows, 128), lambda i: (i, 0))


def _full_spec(shape):
    return pl.BlockSpec(shape, lambda i: tuple(0 for _ in shape))


def _pair2(W):
    z = jnp.zeros(W.shape, W.dtype)
    return jnp.block([[W, z], [z, W]])


def kernel(x, edge_distance, edge_index, params):
    p = params
    f32 = jnp.float32
    x2 = x.reshape(_N, _D)
    dst = edge_index[1]
    src = edge_index[0]
    dst2 = dst.reshape(_E // _ROW, _ROW)
    src2 = src.reshape(_E // _ROW, _ROW)

    Wm0 = p["Wm0"]
    Wm0a = Wm0[:, 64:128]
    lng4 = jnp.tile(p["ln_g"], 8).reshape(1, 128)
    lnb4 = jnp.tile(p["ln_b"], 8).reshape(1, 128)
    adotf = jnp.tile(p["alpha_dot"].reshape(64), 2).reshape(1, 128)
    Gm = jnp.kron(jnp.eye(4, dtype=f32), jnp.ones((16, 16), f32) / 16.0)
    S2P = jnp.zeros((256, 256), f32)
    S2P = S2P.at[0:64, 0:64].set(Gm)
    S2P = S2P.at[128:192, 64:128].set(Gm)
    S2P = S2P.at[64:128, 128:192].set(Gm)
    S2P = S2P.at[192:256, 192:256].set(Gm)
    Ssel = jnp.kron(jnp.eye(4, dtype=f32), jnp.ones((16, 1), f32))
    wrep = jnp.kron(jnp.eye(4, dtype=f32), jnp.ones((1, 16), f32))
    bias_row = jnp.concatenate([p["bp0"], jnp.zeros(48, f32)]).reshape(1, 64)
    Wr1p = jnp.concatenate([p["Wr1"], jnp.zeros((14, 64), f32)], axis=0)
    Wg01 = jnp.concatenate([p["Wg0"], p["Wg1"]], axis=1)
    bg01 = jnp.concatenate([p["bg0"], p["bg1"]])
    W1c = jnp.block([[p["W1r"], p["W1i"]], [-p["W1i"], p["W1r"]]])
    W2c = jnp.block([[p["W2r"], p["W2i"]], [-p["W2i"], p["W2r"]]])
    z = jnp.zeros((64, 16), f32)
    Wpc = jnp.block([
        [p["Wp0"], z, z, z],
        [z, p["Wp1"], z, z],
        [z, z, p["Wp1"], z],
        [z, z, z, p["Wp1"]],
    ])
    Wr1P = _pair2(Wr1p)
    Wr2P = _pair2(p["Wr2"])
    Wg01P = _pair2(Wg01)
    Wm0v = jnp.concatenate([Wm0[:, 0:64], Wm0[:, 128:160]], axis=1)
    Wm0vP = _pair2(Wm0v)
    Wm0aP = _pair2(Wm0a)
    SselP = _pair2(Ssel)
    wrepP = _pair2(wrep)
    W1cP = _pair2(W1c)
    W2m0P = _pair2(p["W2m0"])
    W2cP = _pair2(W2c)
    WpcP = _pair2(Wpc)
    br1P = jnp.tile(p["br1"], 2).reshape(1, 128)
    br2P = jnp.tile(p["br2"], 2).reshape(1, 128)
    bg01P = jnp.tile(bg01, 2).reshape(1, 192)
    edp = jnp.concatenate(
        [edge_distance, jnp.zeros((_E, 14), f32)], axis=1).reshape(
            _E // 2, 128)

    xd, xs = _k0_gather(x2, dst2, src2)
    xdp = xd.reshape(_E // 2, 128)
    xsp = xs.reshape(_E // 2, 128)

    grid = (_E // _EB,)
    w_specs1 = [
        _full_spec((128, 128)), _full_spec((1, 128)),
        _full_spec((128, 128)), _full_spec((1, 128)),
        _full_spec((128, 192)), _full_spec((1, 192)),
        _full_spec((128, 128)), _full_spec((256, 256)),
        _full_spec((1, 128)), _full_spec((1, 128)), _full_spec((1, 128)),
        _full_spec((128, 8)),
    ]
    eap, radp = pl.pallas_call(
        _k1_body,
        grid=grid,
        in_specs=[_blk_spec(_EB // 2), _blk_spec(_EB // 2),
                  _blk_spec(_EB // 2)] + w_specs1,
        out_specs=[pl.BlockSpec((_EB // 2, 32), lambda i: (i, 0)),
                   _blk_spec(_EB // 2)],
        out_shape=[jax.ShapeDtypeStruct((_E // 2, 32), f32),
                   jax.ShapeDtypeStruct((_E // 2, 128), f32)],
    )(edp, xdp, xsp, Wr1P, br1P, Wr2P, br2P, Wg01P, bg01P,
      Wm0aP, S2P, lng4, lnb4, adotf, SselP)
    ea = eap.reshape(_E, 16)

    s0, s1 = _sc_scatter_add(dst2, ea, 16)

    sd = _k2b_gather2(s0, s1, dst2)
    sdp = sd.reshape(_E // 2, 32)

    w_specs3 = [
        _full_spec((128, 192)), _full_spec((1, 192)),
        _full_spec((128, 192)),
        _full_spec((8, 128)), _full_spec((128, 128)),
        _full_spec((128, 256)), _full_spec((128, 256)),
        _full_spec((512, 128)),
    ]
    ppc = pl.pallas_call(
        _k3_body,
        grid=grid,
        in_specs=[_blk_spec(_EB // 2), _blk_spec(_EB // 2),
                  _blk_spec(_EB // 2),
                  pl.BlockSpec((_EB // 2, 32), lambda i: (i, 0)),
                  pl.BlockSpec((_EB // 2, 32), lambda i: (i, 0))] + w_specs3,
        out_specs=_blk_spec(_EB // 2),
        out_shape=jax.ShapeDtypeStruct((_E // 2, 128), f32),
    )(xdp, xsp, radp, eap, sdp, Wg01P, bg01P, Wm0vP,
      wrepP, W1cP, W2m0P, W2cP, WpcP)
    pp = ppc.reshape(_E, _D)

    n0, n1 = _sc_scatter_add(dst2, pp, _D)

    NB = 2000
    out = pl.pallas_call(
        _k5_body,
        grid=(_N // NB,),
        in_specs=[pl.BlockSpec((NB, _D), lambda i: (i, 0)),
                  pl.BlockSpec((NB, _D), lambda i: (i, 0)),
                  _full_spec((1, _D))],
        out_specs=pl.BlockSpec((NB, _D), lambda i: (i, 0)),
        out_shape=jax.ShapeDtypeStruct((_N, _D), f32),
    )(n0, n1, bias_row)

    return out.reshape(_N, 4, 16)


# speedup vs baseline: 1.0445x; 1.0445x over previous
import functools

import jax
import jax.numpy as jnp
from jax import lax
from jax.experimental import pallas as pl
from jax.experimental.pallas import tpu as pltpu
from jax.experimental.pallas import tpu_sc as plsc

_N = 10000
_E = 320000
_D = 64
_NC = 2
_NS = 16
_ROW = 125
_GROUP = 8
_GR = _GROUP * _ROW
_EB = 6400

_mesh = lambda: plsc.VectorSubcoreMesh(core_axis_name="c", subcore_axis_name="s")
_SC_PARAMS = pltpu.CompilerParams(use_tc_tiling_on_sc=False)


def _silu(x):
    return x * jax.nn.sigmoid(x)


def _k0_gather(x2, dst2, src2):
    RPT = _E // _NS
    GP = 4
    GRH = GP * _ROW
    NG = RPT // GRH

    @functools.partial(
        pl.kernel,
        mesh=_mesh(),
        compiler_params=_SC_PARAMS,
        out_type=(
            jax.ShapeDtypeStruct((_E, _D), jnp.float32),
            jax.ShapeDtypeStruct((_E, _D), jnp.float32),
        ),
        scratch_types=[
            pltpu.VMEM((GP, _ROW), jnp.int32),
            pltpu.VMEM((GP, _ROW), jnp.int32),
            pltpu.VMEM((GRH, _D), jnp.float32),
            pltpu.VMEM((GRH, _D), jnp.float32),
            pltpu.SemaphoreType.DMA,
            pltpu.SemaphoreType.DMA,
        ],
    )
    def k(x_hbm, dstr_hbm, srcr_hbm, outd_hbm, outs_hbm, idx0_v, idx1_v,
          rows0_v, rows1_v, sem0, sem1):
        c = lax.axis_index("c")
        s = lax.axis_index("s")
        rbase = s * (RPT // _ROW)

        def run(idx_hbm, out_hbm):
            sets = ((idx0_v, rows0_v, sem0), (idx1_v, rows1_v, sem1))

            def fire(g, st):
                idx_v, rows_v, sem = st
                pltpu.sync_copy(idx_hbm.at[pl.ds(rbase + g * GP, GP)], idx_v)
                for j in range(GP):
                    pltpu.async_copy(x_hbm.at[idx_v.at[j]],
                                     rows_v.at[pl.ds(j * _ROW, _ROW)], sem)

            def drain_write(g, st):
                idx_v, rows_v, sem = st
                for j in range(GP):
                    pltpu.make_async_copy(
                        x_hbm.at[idx_v.at[j]],
                        rows_v.at[pl.ds(j * _ROW, _ROW)], sem).wait()
                pltpu.sync_copy(rows_v,
                                out_hbm.at[pl.ds(s * RPT + g * GRH, GRH)])

            fire(0, sets[0])

            def body(h, carry):
                g0 = 2 * h
                fire(g0 + 1, sets[1])
                drain_write(g0, sets[0])

                @pl.when(g0 + 2 < NG)
                def _():
                    fire(g0 + 2, sets[0])
                drain_write(g0 + 1, sets[1])
                return carry
            lax.fori_loop(0, NG // 2, body, 0)

        @pl.when(c == 0)
        def _():
            run(dstr_hbm, outd_hbm)

        @pl.when(c == 1)
        def _():
            run(srcr_hbm, outs_hbm)

    return k(x2, dst2, src2)


def _sc_scatter_add(dst2, vals, width):
    EPT = _E // (_NC * _NS)
    group = 8 if width <= 16 else 4
    gr = group * _ROW
    NG = EPT // gr
    NZR = _N // _NS
    chunks = []
    off = 0
    while off < NZR:
        sz = min(gr, NZR - off)
        chunks.append((off, sz))
        off += sz

    @functools.partial(
        pl.kernel,
        mesh=_mesh(),
        compiler_params=_SC_PARAMS,
        out_type=(
            jax.ShapeDtypeStruct((_N, width), jnp.float32),
            jax.ShapeDtypeStruct((_N, width), jnp.float32),
        ),
        scratch_types=[
            pltpu.VMEM_SHARED((_N, width), jnp.float32),
            pltpu.VMEM((group, _ROW), jnp.int32),
            pltpu.VMEM((gr, width), jnp.float32),
            pltpu.SemaphoreType.DMA,
        ],
    )
    def k(dst_hbm, vals_hbm, out0_hbm, out1_hbm, acc_sh, idx_v, rows_v, sem):
        c = lax.axis_index("c")
        s = lax.axis_index("s")

        def zero_row(i, carry):
            for kk in range(width // 16):
                rows_v[i, pl.ds(kk * 16, 16)] = jnp.zeros((16,), jnp.float32)
            return carry
        lax.fori_loop(0, min(gr, NZR), zero_row, 0)
        for off, sz in chunks:
            pltpu.sync_copy(rows_v.at[pl.ds(0, sz)],
                            acc_sh.at[pl.ds(s * NZR + off, sz)])
        plsc.subcore_barrier()

        wid = c * _NS + s
        rbase = wid * (EPT // _ROW)

        def body(g, carry):
            pltpu.sync_copy(dst_hbm.at[pl.ds(rbase + g * group, group)],
                            idx_v)
            pltpu.sync_copy(vals_hbm.at[pl.ds(wid * EPT + g * gr, gr)],
                            rows_v)
            descs = []
            for j in range(group):
                descs.append(pltpu.async_copy(
                    rows_v.at[pl.ds(j * _ROW, _ROW)],
                    acc_sh.at[idx_v.at[j]], sem, add=True))
            for d in descs:
                d.wait()
            return carry
        lax.fori_loop(0, NG, body, 0)
        plsc.subcore_barrier()

        for off, sz in chunks:
            pltpu.sync_copy(acc_sh.at[pl.ds(s * NZR + off, sz)],
                            rows_v.at[pl.ds(0, sz)])

            @pl.when(c == 0)
            def _(off=off, sz=sz):
                pltpu.sync_copy(rows_v.at[pl.ds(0, sz)],
                                out0_hbm.at[pl.ds(s * NZR + off, sz)])

            @pl.when(c == 1)
            def _(off=off, sz=sz):
                pltpu.sync_copy(rows_v.at[pl.ds(0, sz)],
                                out1_hbm.at[pl.ds(s * NZR + off, sz)])

    return k(dst2, vals)


def _k2b_gather2(s0, s1, dst2):
    EPT = _E // (_NC * _NS)
    NG = EPT // _GR

    @functools.partial(
        pl.kernel,
        mesh=_mesh(),
        compiler_params=_SC_PARAMS,
        out_type=jax.ShapeDtypeStruct((_E, 16), jnp.float32),
        scratch_types=[
            pltpu.VMEM((_GROUP, _ROW), jnp.int32),
            pltpu.VMEM((_GR, 16), jnp.float32),
            pltpu.VMEM((_GR, 16), jnp.float32),
            pltpu.SemaphoreType.DMA,
        ],
    )
    def k(s0_hbm, s1_hbm, dstr_hbm, o_hbm, idx_v, r0_v, r1_v, sem):
        c = lax.axis_index("c")
        s = lax.axis_index("s")
        wid = c * _NS + s
        rbase = wid * (EPT // _ROW)

        def body(g, carry):
            pltpu.sync_copy(dstr_hbm.at[pl.ds(rbase + g * _GROUP, _GROUP)],
                            idx_v)
            descs = []
            for j in range(_GROUP):
                descs.append(pltpu.async_copy(
                    s0_hbm.at[idx_v.at[j]],
                    r0_v.at[pl.ds(j * _ROW, _ROW)], sem))
                descs.append(pltpu.async_copy(
                    s1_hbm.at[idx_v.at[j]],
                    r1_v.at[pl.ds(j * _ROW, _ROW)], sem))
            for d in descs:
                d.wait()

            def addrow(i, carry2):
                for kk in range(4):
                    row = i * 4 + kk
                    r0_v[row, :] = r0_v[row, :] + r1_v[row, :]
                return carry2
            lax.fori_loop(0, _GR // 4, addrow, 0)
            pltpu.sync_copy(r0_v, o_hbm.at[pl.ds(wid * EPT + g * _GR, _GR)])
            return carry
        lax.fori_loop(0, NG, body, 0)

    return k(s0, s1, dst2)


def _dot(a, b):
    return jnp.dot(a, b, preferred_element_type=jnp.float32)


def _cat(*xs):
    return jnp.concatenate(xs, axis=1)


def _m0_in_paired(xdP, xsP):
    return _cat(xdP[:, 0:16], xsP[:, 0:16], xdP[:, 32:48], xsP[:, 32:48],
                xdP[:, 64:80], xsP[:, 64:80], xdP[:, 96:112], xsP[:, 96:112])


def _k1_body(edP, xdP, xsP, Wr1P, br1P, Wr2P, br2P, Wg01P, bg01P, Wm0aP,
             S2P, lng4P, lnb4P, adotP, SselP, ea_ref, rad_ref):
    h = _silu(_dot(edP[...], Wr1P[...]) + br1P[...])
    radP = _silu(_dot(h, Wr2P[...]) + br2P[...])
    rad_ref[...] = radP
    g01P = _dot(radP, Wg01P[...]) + bg01P[...]
    gate0P = _cat(g01P[:, 0:64], g01P[:, 96:160])
    tP = _m0_in_paired(xdP[...], xsP[...]) * gate0P
    a1P = _dot(tP, Wm0aP[...])
    statsP = _dot(_cat(a1P, a1P * a1P), S2P[...])
    muP = _cat(statsP[:, 0:64], statsP[:, 128:192])
    msqP = _cat(statsP[:, 64:128], statsP[:, 192:256])
    varP = msqP - muP * muP
    xnP = (a1P - muP) * lax.rsqrt(varP + 1e-5) * lng4P[...] + lnb4P[...]
    slP = 0.8 * xnP * jax.nn.sigmoid(xnP) + 0.2 * xnP
    eaP = jnp.exp(_dot(slP * adotP[...], SselP[...]))
    z12 = jnp.zeros((eaP.shape[0], 12), jnp.float32)
    ea_ref[...] = _cat(eaP[:, 0:4], z12, eaP[:, 4:8], z12)


def _k3_body(xdP, xsP, radP, eab, sdP, Wg01P, bg01P, Wm0vP, wrepP, W1cP,
             W2m0P, W2cP, WpcP, out_ref):
    xdP_, xsP_, sdP_, eab_ = xdP[...], xsP[...], sdP[...], eab[...]
    g01P = _dot(radP[...], Wg01P[...]) + bg01P[...]
    gate0P = _cat(g01P[:, 0:64], g01P[:, 96:160])
    tP = _m0_in_paired(xdP_, xsP_) * gate0P
    m0fP = _dot(tP, Wm0vP[...])
    eaP = _cat(eab_[:, 0:4], eab_[:, 16:20])
    w = eaP / (_cat(sdP_[:, 0:4], sdP_[:, 16:20]) + 1e-9)
    wtP = _dot(w, wrepP[...])
    m01P = _cat(m0fP[:, 32:64], m0fP[:, 128:160])
    gP = _cat(m0fP[:, 64:96], m0fP[:, 160:192])
    sgP = jax.nn.sigmoid(gP)
    g1e0 = g01P[:, 64:96]
    g1e1 = g01P[:, 160:192]
    rimP = _cat(xdP_[:, 16:32], xsP_[:, 16:32], xdP_[:, 48:64],
                xsP_[:, 48:64], xdP_[:, 80:96], xsP_[:, 80:96],
                xdP_[:, 112:128], xsP_[:, 112:128]) * _cat(
                    g1e0, g1e0, g1e1, g1e1)
    o1P = _dot(rimP, W1cP[...])
    m0in2P = _cat(gP[:, 0:32] * sgP[:, 0:32], m01P[:, 0:32] * sgP[:, 0:32],
                  gP[:, 32:64] * sgP[:, 32:64],
                  m01P[:, 32:64] * sgP[:, 32:64])
    m0o2P = _dot(m0in2P, W2m0P[...])
    r2i2P = _cat(o1P[:, 0:32] * sgP[:, 0:32], o1P[:, 32:64] * sgP[:, 0:32],
                 o1P[:, 64:96] * sgP[:, 32:64],
                 o1P[:, 96:128] * sgP[:, 32:64])
    ocP = _dot(r2i2P, W2cP[...])
    w0 = wtP[:, 0:64]
    w1 = wtP[:, 64:128]
    vcatP = _cat(m0o2P[:, 0:64] * w0, ocP[:, 0:64] * w0,
                 m0o2P[:, 64:128] * w0, ocP[:, 64:128] * w0,
                 m0o2P[:, 128:192] * w1, ocP[:, 128:192] * w1,
                 m0o2P[:, 192:256] * w1, ocP[:, 192:256] * w1)
    out_ref[...] = _dot(vcatP, WpcP[...])


def _k5_body(n0, n1, bias, out_ref):
    out_ref[...] = n0[...] + n1[...] + bias[...]


def _blk_spec(rows):
    return pl.BlockSpec((rows, 128), lambda i: (i, 0))


def _full_spec(shape):
    return pl.BlockSpec(shape, lambda i: tuple(0 for _ in shape))


def _pair2(W):
    z = jnp.zeros(W.shape, W.dtype)
    return jnp.block([[W, z], [z, W]])


def kernel(x, edge_distance, edge_index, params):
    p = params
    f32 = jnp.float32
    x2 = x.reshape(_N, _D)
    dst = edge_index[1]
    src = edge_index[0]
    dst2 = dst.reshape(_E // _ROW, _ROW)
    src2 = src.reshape(_E // _ROW, _ROW)

    Wm0 = p["Wm0"]
    Wm0a = Wm0[:, 64:128]
    lng4 = jnp.tile(p["ln_g"], 8).reshape(1, 128)
    lnb4 = jnp.tile(p["ln_b"], 8).reshape(1, 128)
    adotf = jnp.tile(p["alpha_dot"].reshape(64), 2).reshape(1, 128)
    Gm = jnp.kron(jnp.eye(4, dtype=f32), jnp.ones((16, 16), f32) / 16.0)
    S2P = jnp.zeros((256, 256), f32)
    S2P = S2P.at[0:64, 0:64].set(Gm)
    S2P = S2P.at[128:192, 64:128].set(Gm)
    S2P = S2P.at[64:128, 128:192].set(Gm)
    S2P = S2P.at[192:256, 192:256].set(Gm)
    Ssel = jnp.kron(jnp.eye(4, dtype=f32), jnp.ones((16, 1), f32))
    wrep = jnp.kron(jnp.eye(4, dtype=f32), jnp.ones((1, 16), f32))
    bias_row = jnp.concatenate([p["bp0"], jnp.zeros(48, f32)]).reshape(1, 64)
    Wr1p = jnp.concatenate([p["Wr1"], jnp.zeros((14, 64), f32)], axis=0)
    Wg01 = jnp.concatenate([p["Wg0"], p["Wg1"]], axis=1)
    bg01 = jnp.concatenate([p["bg0"], p["bg1"]])
    W1c = jnp.block([[p["W1r"], p["W1i"]], [-p["W1i"], p["W1r"]]])
    W2c = jnp.block([[p["W2r"], p["W2i"]], [-p["W2i"], p["W2r"]]])
    z = jnp.zeros((64, 16), f32)
    Wpc = jnp.block([
        [p["Wp0"], z, z, z],
        [z, p["Wp1"], z, z],
        [z, z, p["Wp1"], z],
        [z, z, z, p["Wp1"]],
    ])
    Wr1P = _pair2(Wr1p)
    Wr2P = _pair2(p["Wr2"])
    Wg01P = _pair2(Wg01)
    Wm0v = jnp.concatenate([Wm0[:, 0:64], Wm0[:, 128:160]], axis=1)
    Wm0vP = _pair2(Wm0v)
    Wm0aP = _pair2(Wm0a)
    SselP = _pair2(Ssel)
    wrepP = _pair2(wrep)
    W1cP = _pair2(W1c)
    W2m0P = _pair2(p["W2m0"])
    W2cP = _pair2(W2c)
    WpcP = _pair2(Wpc)
    br1P = jnp.tile(p["br1"], 2).reshape(1, 128)
    br2P = jnp.tile(p["br2"], 2).reshape(1, 128)
    bg01P = jnp.tile(bg01, 2).reshape(1, 192)
    edp = jnp.concatenate(
        [edge_distance, jnp.zeros((_E, 14), f32)], axis=1).reshape(
            _E // 2, 128)

    xd, xs = _k0_gather(x2, dst2, src2)
    xdp = xd.reshape(_E // 2, 128)
    xsp = xs.reshape(_E // 2, 128)

    grid = (_E // _EB,)
    w_specs1 = [
        _full_spec((128, 128)), _full_spec((1, 128)),
        _full_spec((128, 128)), _full_spec((1, 128)),
        _full_spec((128, 192)), _full_spec((1, 192)),
        _full_spec((128, 128)), _full_spec((256, 256)),
        _full_spec((1, 128)), _full_spec((1, 128)), _full_spec((1, 128)),
        _full_spec((128, 8)),
    ]
    eap, radp = pl.pallas_call(
        _k1_body,
        grid=grid,
        in_specs=[_blk_spec(_EB // 2), _blk_spec(_EB // 2),
                  _blk_spec(_EB // 2)] + w_specs1,
        out_specs=[pl.BlockSpec((_EB // 2, 32), lambda i: (i, 0)),
                   _blk_spec(_EB // 2)],
        out_shape=[jax.ShapeDtypeStruct((_E // 2, 32), f32),
                   jax.ShapeDtypeStruct((_E // 2, 128), f32)],
    )(edp, xdp, xsp, Wr1P, br1P, Wr2P, br2P, Wg01P, bg01P,
      Wm0aP, S2P, lng4, lnb4, adotf, SselP)
    ea = eap.reshape(_E, 16)

    s0, s1 = _sc_scatter_add(dst2, ea, 16)

    sd = _k2b_gather2(s0, s1, dst2)
    sdp = sd.reshape(_E // 2, 32)

    w_specs3 = [
        _full_spec((128, 192)), _full_spec((1, 192)),
        _full_spec((128, 192)),
        _full_spec((8, 128)), _full_spec((128, 128)),
        _full_spec((128, 256)), _full_spec((128, 256)),
        _full_spec((512, 128)),
    ]
    ppc = pl.pallas_call(
        _k3_body,
        grid=grid,
        in_specs=[_blk_spec(_EB // 2), _blk_spec(_EB // 2),
                  _blk_spec(_EB // 2),
                  pl.BlockSpec((_EB // 2, 32), lambda i: (i, 0)),
                  pl.BlockSpec((_EB // 2, 32), lambda i: (i, 0))] + w_specs3,
        out_specs=_blk_spec(_EB // 2),
        out_shape=jax.ShapeDtypeStruct((_E // 2, 128), f32),
    )(xdp, xsp, radp, eap, sdp, Wg01P, bg01P, Wm0vP,
      wrepP, W1cP, W2m0P, W2cP, WpcP)
    pp = ppc.reshape(_E, _D)

    n0, n1 = _sc_scatter_add(dst2, pp, _D)

    NB = 2000
    out = pl.pallas_call(
        _k5_body,
        grid=(_N // NB,),
        in_specs=[pl.BlockSpec((NB, _D), lambda i: (i, 0)),
                  pl.BlockSpec((NB, _D), lambda i: (i, 0)),
                  _full_spec((1, _D))],
        out_specs=pl.BlockSpec((NB, _D), lambda i: (i, 0)),
        out_shape=jax.ShapeDtypeStruct((_N, _D), f32),
    )(n0, n1, bias_row)

    return out.reshape(_N, 4, 16)
